# baseline (speedup 1.0000x reference)
import functools

import jax
import jax.numpy as jnp
from jax import lax
from jax.experimental import pallas as pl
from jax.experimental.pallas import tpu as pltpu
from jax.experimental.pallas import tpu_sc as plsc

_NUM_CORES = 2
_NUM_SUBCORES = 16
_NW = _NUM_CORES * _NUM_SUBCORES
_CHUNK = 16


def _make_gather(total_pad: int, per_w: int, n_chunk: int, d: int):
    mesh = plsc.VectorSubcoreMesh(core_axis_name="c", subcore_axis_name="s")

    @functools.partial(
        pl.kernel,
        mesh=mesh,
        out_type=jax.ShapeDtypeStruct((total_pad, d), jnp.float32),
        scratch_types=[
            pltpu.VMEM((n_chunk, _CHUNK), jnp.int32),
            pltpu.VMEM((_CHUNK, d), jnp.float32),
            pltpu.VMEM((_CHUNK, d), jnp.float32),
            pltpu.SemaphoreType.DMA,
            pltpu.SemaphoreType.DMA,
            pltpu.SemaphoreType.DMA,
            pltpu.SemaphoreType.DMA,
        ],
    )
    def gather_kernel(table_hbm, idx_hbm, out_hbm, idx_v, rows0, rows1,
                      gsem0, gsem1, osem0, osem1):
        wid = lax.axis_index("s") * _NUM_CORES + lax.axis_index("c")
        base = wid * per_w
        pltpu.sync_copy(idx_hbm.at[wid], idx_v)

        rows = (rows0, rows1)
        gsems = (gsem0, gsem1)
        osems = (osem0, osem1)

        pltpu.make_async_copy(table_hbm.at[idx_v.at[0]], rows0, gsem0).start()

        for c in range(n_chunk):
            cur = c % 2
            nxt = (c + 1) % 2
            pltpu.make_async_copy(
                table_hbm.at[idx_v.at[c]], rows[cur], gsems[cur]).wait()
            if c + 1 < n_chunk:
                if c >= 1:
                    pltpu.make_async_copy(
                        rows[nxt],
                        out_hbm.at[pl.ds((c - 1) * _CHUNK + base, _CHUNK)],
                        osems[nxt]).wait()
                pltpu.make_async_copy(
                    table_hbm.at[idx_v.at[c + 1]], rows[nxt],
                    gsems[nxt]).start()
            pltpu.make_async_copy(
                rows[cur], out_hbm.at[pl.ds(c * _CHUNK + base, _CHUNK)],
                osems[cur]).start()
        last = n_chunk - 1
        if n_chunk >= 2:
            pltpu.make_async_copy(
                rows[(last - 1) % 2],
                out_hbm.at[pl.ds((last - 1) * _CHUNK + base, _CHUNK)],
                osems[(last - 1) % 2]).wait()
        pltpu.make_async_copy(
            rows[last % 2], out_hbm.at[pl.ds(last * _CHUNK + base, _CHUNK)],
            osems[last % 2]).wait()

    return gather_kernel


def kernel(last_hidden_state, offsets, mask):
    del mask
    b, t, d = last_hidden_state.shape
    n = offsets.shape[1]
    r = n - 2
    total = b * r

    starts = offsets[:, 1 : n - 1, 0]
    flat_idx = (starts + (jnp.arange(b, dtype=jnp.int32) * t)[:, None]).reshape(-1)

    per_w = -(-total // _NW)
    per_w = -(-per_w // _CHUNK) * _CHUNK
    total_pad = per_w * _NW
    n_chunk = per_w // _CHUNK
    flat_idx = jnp.pad(flat_idx, (0, total_pad - total))
    idx3 = flat_idx.reshape(_NW, n_chunk, _CHUNK)

    table = last_hidden_state.reshape(b * t, d)
    out = _make_gather(total_pad, per_w, n_chunk, d)(table, idx3)
    return out[:total].reshape(b, r, d)


# speedup vs baseline: 1.2417x; 1.2417x over previous
import functools

import jax
import jax.numpy as jnp
from jax import lax
from jax.experimental import pallas as pl
from jax.experimental.pallas import tpu as pltpu
from jax.experimental.pallas import tpu_sc as plsc

_NUM_CORES = 2
_NUM_SUBCORES = 16
_NW = _NUM_CORES * _NUM_SUBCORES
_CHUNK = 16


def _make_gather(total: int, per_w: int, n_chunk: int, d: int):
    mesh = plsc.VectorSubcoreMesh(core_axis_name="c", subcore_axis_name="s")

    @functools.partial(
        pl.kernel,
        mesh=mesh,
        out_type=jax.ShapeDtypeStruct((total, d), jnp.float32),
        scratch_types=[
            pltpu.VMEM((n_chunk, _CHUNK), jnp.int32),
            pltpu.VMEM((_CHUNK, d), jnp.float32),
            pltpu.VMEM((_CHUNK, d), jnp.float32),
            pltpu.SemaphoreType.DMA,
            pltpu.SemaphoreType.DMA,
            pltpu.SemaphoreType.DMA,
            pltpu.SemaphoreType.DMA,
        ],
    )
    def gather_kernel(table_hbm, idx_hbm, out_hbm, idx_v, rows0, rows1,
                      gsem0, gsem1, osem0, osem1):
        wid = lax.axis_index("s") * _NUM_CORES + lax.axis_index("c")
        base = jnp.where(wid == _NW - 1, total - per_w, wid * per_w)
        pltpu.sync_copy(idx_hbm.at[wid], idx_v)

        rows = (rows0, rows1)
        gsems = (gsem0, gsem1)
        osems = (osem0, osem1)

        pltpu.make_async_copy(table_hbm.at[idx_v.at[0]], rows0, gsem0).start()

        for c in range(n_chunk):
            cur = c % 2
            nxt = (c + 1) % 2
            pltpu.make_async_copy(
                table_hbm.at[idx_v.at[c]], rows[cur], gsems[cur]).wait()
            if c + 1 < n_chunk:
                if c >= 1:
                    pltpu.make_async_copy(
                        rows[nxt],
                        out_hbm.at[pl.ds((c - 1) * _CHUNK + base, _CHUNK)],
                        osems[nxt]).wait()
                pltpu.make_async_copy(
                    table_hbm.at[idx_v.at[c + 1]], rows[nxt],
                    gsems[nxt]).start()
            pltpu.make_async_copy(
                rows[cur], out_hbm.at[pl.ds(c * _CHUNK + base, _CHUNK)],
                osems[cur]).start()
        last = n_chunk - 1
        if n_chunk >= 2:
            pltpu.make_async_copy(
                rows[(last - 1) % 2],
                out_hbm.at[pl.ds((last - 1) * _CHUNK + base, _CHUNK)],
                osems[(last - 1) % 2]).wait()
        pltpu.make_async_copy(
            rows[last % 2], out_hbm.at[pl.ds(last * _CHUNK + base, _CHUNK)],
            osems[last % 2]).wait()

    return gather_kernel


def kernel(last_hidden_state, offsets, mask):
    del mask
    b, t, d = last_hidden_state.shape
    n = offsets.shape[1]
    r = n - 2
    total = b * r

    starts = offsets[:, 1 : n - 1, 0]
    flat_idx = (starts + (jnp.arange(b, dtype=jnp.int32) * t)[:, None]).reshape(-1)

    per_w = -(-total // _NW)
    per_w = -(-per_w // _CHUNK) * _CHUNK
    n_chunk = per_w // _CHUNK
    tail_base = total - per_w
    assert tail_base % 8 == 0 and tail_base >= 0
    idx_blocks = jnp.concatenate(
        [flat_idx[: (_NW - 1) * per_w], flat_idx[tail_base:]])
    idx3 = idx_blocks.reshape(_NW, n_chunk, _CHUNK)

    table = last_hidden_state.reshape(b * t, d)
    out = _make_gather(total, per_w, n_chunk, d)(table, idx3)
    return out.reshape(b, r, d)


# speedup vs baseline: 1.2439x; 1.0017x over previous
import functools

import jax
import jax.numpy as jnp
from jax import lax
from jax.experimental import pallas as pl
from jax.experimental.pallas import tpu as pltpu
from jax.experimental.pallas import tpu_sc as plsc

_NUM_CORES = 2
_NUM_SUBCORES = 16
_NW = _NUM_CORES * _NUM_SUBCORES
_CHUNK = 16


def _make_gather(total: int, per_w: int, n_chunk: int, d: int):
    mesh = plsc.VectorSubcoreMesh(core_axis_name="c", subcore_axis_name="s")

    @functools.partial(
        pl.kernel,
        mesh=mesh,
        out_type=jax.ShapeDtypeStruct((total, d), jnp.float32),
        compiler_params=pltpu.CompilerParams(use_tc_tiling_on_sc=True),
        scratch_types=[
            pltpu.VMEM((n_chunk, _CHUNK), jnp.int32),
            pltpu.VMEM((_CHUNK, d), jnp.float32),
            pltpu.VMEM((_CHUNK, d), jnp.float32),
            pltpu.SemaphoreType.DMA,
            pltpu.SemaphoreType.DMA,
            pltpu.SemaphoreType.DMA,
            pltpu.SemaphoreType.DMA,
        ],
    )
    def gather_kernel(table_hbm, idx_hbm, out_hbm, idx_v, rows0, rows1,
                      gsem0, gsem1, osem0, osem1):
        wid = lax.axis_index("s") * _NUM_CORES + lax.axis_index("c")
        base = jnp.where(wid == _NW - 1, total - per_w, wid * per_w)
        pltpu.sync_copy(idx_hbm.at[wid], idx_v)

        rows = (rows0, rows1)
        gsems = (gsem0, gsem1)
        osems = (osem0, osem1)

        pltpu.make_async_copy(table_hbm.at[idx_v.at[0]], rows0, gsem0).start()

        for c in range(n_chunk):
            cur = c % 2
            nxt = (c + 1) % 2
            pltpu.make_async_copy(
                table_hbm.at[idx_v.at[c]], rows[cur], gsems[cur]).wait()
            if c + 1 < n_chunk:
                if c >= 1:
                    pltpu.make_async_copy(
                        rows[nxt],
                        out_hbm.at[pl.ds((c - 1) * _CHUNK + base, _CHUNK)],
                        osems[nxt]).wait()
                pltpu.make_async_copy(
                    table_hbm.at[idx_v.at[c + 1]], rows[nxt],
                    gsems[nxt]).start()
            pltpu.make_async_copy(
                rows[cur], out_hbm.at[pl.ds(c * _CHUNK + base, _CHUNK)],
                osems[cur]).start()
        last = n_chunk - 1
        if n_chunk >= 2:
            pltpu.make_async_copy(
                rows[(last - 1) % 2],
                out_hbm.at[pl.ds((last - 1) * _CHUNK + base, _CHUNK)],
                osems[(last - 1) % 2]).wait()
        pltpu.make_async_copy(
            rows[last % 2], out_hbm.at[pl.ds(last * _CHUNK + base, _CHUNK)],
            osems[last % 2]).wait()

    return gather_kernel


def kernel(last_hidden_state, offsets, mask):
    del mask
    b, t, d = last_hidden_state.shape
    n = offsets.shape[1]
    r = n - 2
    total = b * r

    starts = offsets[:, 1 : n - 1, 0]
    flat_idx = (starts + (jnp.arange(b, dtype=jnp.int32) * t)[:, None]).reshape(-1)

    per_w = -(-total // _NW)
    per_w = -(-per_w // _CHUNK) * _CHUNK
    n_chunk = per_w // _CHUNK
    tail_base = total - per_w
    assert tail_base % 8 == 0 and tail_base >= 0
    idx_blocks = jnp.concatenate(
        [flat_idx[: (_NW - 1) * per_w], flat_idx[tail_base:]])
    idx3 = idx_blocks.reshape(_NW, n_chunk, _CHUNK)

    table = last_hidden_state.reshape(b * t, d)
    out = _make_gather(total, per_w, n_chunk, d)(table, idx3)
    return out.reshape(b, r, d)


# speedup vs baseline: 1.4618x; 1.1752x over previous
import functools

import jax
import jax.numpy as jnp
from jax import lax
from jax.experimental import pallas as pl
from jax.experimental.pallas import tpu as pltpu
from jax.experimental.pallas import tpu_sc as plsc

_NUM_CORES = 2
_NUM_SUBCORES = 16
_NW = _NUM_CORES * _NUM_SUBCORES
_CHUNK = 8
_LANES = 128
_VREG = 16


def _make_gather(total_q: int, per_w: int, n_chunk: int, d: int):
    mesh = plsc.VectorSubcoreMesh(core_axis_name="c", subcore_axis_name="s")
    pieces = d // _LANES
    n_pair = n_chunk // 2

    @functools.partial(
        pl.kernel,
        mesh=mesh,
        out_type=jax.ShapeDtypeStruct((total_q, _LANES), jnp.float32),
        scratch_types=[
            pltpu.VMEM((n_chunk, _CHUNK), jnp.int32),
            pltpu.VMEM((n_chunk, _CHUNK * pieces), jnp.int32),
            pltpu.VMEM((_CHUNK, d), jnp.float32),
            pltpu.VMEM((_CHUNK, d), jnp.float32),
            pltpu.VMEM((_CHUNK * pieces, _LANES), jnp.float32),
            pltpu.VMEM((_CHUNK * pieces, _LANES), jnp.float32),
            pltpu.SemaphoreType.DMA,
            pltpu.SemaphoreType.DMA,
            pltpu.SemaphoreType.DMA,
            pltpu.SemaphoreType.DMA,
        ],
    )
    def gather_kernel(table_hbm, gidx_hbm, qidx_hbm, out_hbm, gidx_v, qidx_v,
                      rows0, rows1, s0, s1, gsem0, gsem1, osem0, osem1):
        wid = lax.axis_index("s") * _NUM_CORES + lax.axis_index("c")
        pltpu.sync_copy(gidx_hbm.at[wid], gidx_v)
        pltpu.sync_copy(qidx_hbm.at[wid], qidx_v)

        def gather(c, buf, sem):
            return pltpu.make_async_copy(table_hbm.at[gidx_v.at[c]], buf, sem)

        def scatter(c, buf, sem):
            return pltpu.make_async_copy(buf, out_hbm.at[qidx_v.at[c]], sem)

        def interleave(src, dst):
            def body(i, carry):
                for k in range(pieces):
                    for v in range(_LANES // _VREG):
                        dst[i * pieces + k, pl.ds(v * _VREG, _VREG)] = (
                            src[i, pl.ds(k * _LANES + v * _VREG, _VREG)])
                return carry

            lax.fori_loop(0, _CHUNK, body, 0)

        gather(0, rows0, gsem0).start()
        gather(1, rows1, gsem1).start()

        def pair(m, carry):
            c0 = m * 2
            c1 = c0 + 1
            for c, rows, s, gsem, osem in (
                (c0, rows0, s0, gsem0, osem0),
                (c1, rows1, s1, gsem1, osem1),
            ):
                gather(c, rows, gsem).wait()

                @pl.when(m >= 1)
                def _wait_prev(c=c, s=s, osem=osem):
                    scatter(c - 2, s, osem).wait()

                interleave(rows, s)
                scatter(c, s, osem).start()

                @pl.when(m < n_pair - 1)
                def _next_gather(c=c, rows=rows, gsem=gsem):
                    gather(c + 2, rows, gsem).start()
            return carry

        lax.fori_loop(0, n_pair, pair, 0)
        scatter(n_chunk - 2, s0, osem0).wait()
        scatter(n_chunk - 1, s1, osem1).wait()

    return gather_kernel


def kernel(last_hidden_state, offsets, mask):
    del mask
    b, t, d = last_hidden_state.shape
    n = offsets.shape[1]
    r = n - 2
    total_g = b * r
    pieces = d // _LANES
    total_q = total_g * pieces

    starts = offsets[:, 1 : n - 1, 0]
    src = (starts + (jnp.arange(b, dtype=jnp.int32) * t)[:, None]).T.reshape(-1)

    per_w = -(-total_g // _NW)
    per_w = -(-per_w // (2 * _CHUNK)) * (2 * _CHUNK)
    n_chunk = per_w // _CHUNK
    tail = total_g - per_w
    assert tail % 8 == 0 and tail >= 0
    g_ids = jnp.concatenate(
        [jnp.arange((_NW - 1) * per_w, dtype=jnp.int32),
         jnp.arange(tail, total_g, dtype=jnp.int32)])
    gidx = jnp.concatenate(
        [src[: (_NW - 1) * per_w], src[tail:]]).reshape(_NW, n_chunk, _CHUNK)
    qbase = (g_ids // b) * (b * pieces) + (g_ids % b)
    qidx = qbase[:, None] + b * jnp.arange(pieces, dtype=jnp.int32)
    qidx = qidx.reshape(_NW, n_chunk, _CHUNK * pieces)

    table = last_hidden_state.reshape(b * t, d)
    out = _make_gather(total_q, per_w, n_chunk, d)(table, gidx, qidx)
    return (out.reshape(r, pieces, b, _LANES)
            .transpose(2, 0, 1, 3)
            .reshape(b, r, d))


# speedup vs baseline: 2.3555x; 1.6114x over previous
import functools

import jax
import jax.numpy as jnp
from jax import lax
from jax.experimental import pallas as pl
from jax.experimental.pallas import tpu as pltpu
from jax.experimental.pallas import tpu_sc as plsc

_NUM_CORES = 2
_NUM_SUBCORES = 16
_NW = _NUM_CORES * _NUM_SUBCORES
_CHUNK = 8
_LANES = 128
_VREG = 16


def _make_gather(total_q: int, per_w: int, n_chunk: int, d: int):
    mesh = plsc.VectorSubcoreMesh(core_axis_name="c", subcore_axis_name="s")
    pieces = d // _LANES
    n_pair = n_chunk // 2

    @functools.partial(
        pl.kernel,
        mesh=mesh,
        out_type=jax.ShapeDtypeStruct((total_q, _LANES), jnp.float32),
        scratch_types=[
            pltpu.VMEM((n_chunk, _CHUNK), jnp.int32),
            pltpu.VMEM((n_chunk, _CHUNK * pieces), jnp.int32),
            pltpu.VMEM((_CHUNK, d), jnp.float32),
            pltpu.VMEM((_CHUNK, d), jnp.float32),
            pltpu.VMEM((_CHUNK * pieces, _LANES), jnp.float32),
            pltpu.VMEM((_CHUNK * pieces, _LANES), jnp.float32),
            pltpu.SemaphoreType.DMA,
            pltpu.SemaphoreType.DMA,
            pltpu.SemaphoreType.DMA,
            pltpu.SemaphoreType.DMA,
        ],
    )
    def gather_kernel(table_hbm, gidx_hbm, qidx_hbm, out_hbm, gidx_v, qidx_v,
                      rows0, rows1, s0, s1, gsem0, gsem1, osem0, osem1):
        wid = lax.axis_index("s") * _NUM_CORES + lax.axis_index("c")
        pltpu.sync_copy(gidx_hbm.at[wid], gidx_v)
        pltpu.sync_copy(qidx_hbm.at[wid], qidx_v)

        def gather(c, buf, sem):
            return pltpu.make_async_copy(table_hbm.at[gidx_v.at[c]], buf, sem)

        def scatter(c, buf, sem):
            return pltpu.make_async_copy(buf, out_hbm.at[qidx_v.at[c]], sem)

        def interleave(src, dst):
            for i in range(_CHUNK):
                for k in range(pieces):
                    for v in range(_LANES // _VREG):
                        dst[i * pieces + k, pl.ds(v * _VREG, _VREG)] = (
                            src[i, pl.ds(k * _LANES + v * _VREG, _VREG)])

        gather(0, rows0, gsem0).start()
        gather(1, rows1, gsem1).start()

        def pair(m, carry):
            c0 = m * 2
            c1 = c0 + 1
            for c, rows, s, gsem, osem in (
                (c0, rows0, s0, gsem0, osem0),
                (c1, rows1, s1, gsem1, osem1),
            ):
                gather(c, rows, gsem).wait()

                @pl.when(m >= 1)
                def _wait_prev(c=c, s=s, osem=osem):
                    scatter(c - 2, s, osem).wait()

                interleave(rows, s)
                scatter(c, s, osem).start()

                @pl.when(m < n_pair - 1)
                def _next_gather(c=c, rows=rows, gsem=gsem):
                    gather(c + 2, rows, gsem).start()
            return carry

        lax.fori_loop(0, n_pair, pair, 0)
        scatter(n_chunk - 2, s0, osem0).wait()
        scatter(n_chunk - 1, s1, osem1).wait()

    return gather_kernel


def kernel(last_hidden_state, offsets, mask):
    del mask
    b, t, d = last_hidden_state.shape
    n = offsets.shape[1]
    r = n - 2
    total_g = b * r
    pieces = d // _LANES
    total_q = total_g * pieces

    starts = offsets[:, 1 : n - 1, 0]
    src = (starts + (jnp.arange(b, dtype=jnp.int32) * t)[:, None]).T.reshape(-1)

    per_w = -(-total_g // _NW)
    per_w = -(-per_w // (2 * _CHUNK)) * (2 * _CHUNK)
    n_chunk = per_w // _CHUNK
    tail = total_g - per_w
    assert tail % 8 == 0 and tail >= 0
    g_ids = jnp.concatenate(
        [jnp.arange((_NW - 1) * per_w, dtype=jnp.int32),
         jnp.arange(tail, total_g, dtype=jnp.int32)])
    gidx = jnp.concatenate(
        [src[: (_NW - 1) * per_w], src[tail:]]).reshape(_NW, n_chunk, _CHUNK)
    qbase = (g_ids // b) * (b * pieces) + (g_ids % b)
    qidx = qbase[:, None] + b * jnp.arange(pieces, dtype=jnp.int32)
    qidx = qidx.reshape(_NW, n_chunk, _CHUNK * pieces)

    table = last_hidden_state.reshape(b * t, d)
    out = _make_gather(total_q, per_w, n_chunk, d)(table, gidx, qidx)
    return (out.reshape(r, pieces, b, _LANES)
            .transpose(2, 0, 1, 3)
            .reshape(b, r, d))
